# baseline (speedup 1.0000x reference)
import functools

import jax
import jax.numpy as jnp
from jax import lax
from jax.experimental import pallas as pl
from jax.experimental.pallas import tpu as pltpu
from jax.experimental.pallas import tpu_sc as plsc

BATCH = 4
SEQ = 2048
D_MODEL = 768

NUM_CORES = 2
NUM_SUBCORES = 16
NUM_WORKERS = NUM_CORES * NUM_SUBCORES
B_PER_W = BATCH * SEQ // NUM_WORKERS
W_PER_BATCH = SEQ // B_PER_W
CHUNK = 32
GROUP = 2
G_ROWS = GROUP * CHUNK
N_GROUPS = B_PER_W // G_ROWS

_mesh = plsc.VectorSubcoreMesh(core_axis_name="c", subcore_axis_name="s")


@functools.partial(
    pl.kernel,
    mesh=_mesh,
    out_type=jax.ShapeDtypeStruct((BATCH, SEQ, D_MODEL), jnp.float32),
    scratch_types=[
        pltpu.VMEM((B_PER_W,), jnp.int32),
        pltpu.VMEM((2, G_ROWS, D_MODEL), jnp.float32),
        pltpu.SemaphoreType.DMA,
        pltpu.SemaphoreType.DMA,
        pltpu.SemaphoreType.DMA,
        pltpu.SemaphoreType.DMA,
    ],
)
def _emb_lookup(
    idx_hbm, table_hbm, out_hbm, idx_v, rows_v, gsem, wsem0, wsem1, isem
):
    wid = lax.axis_index("s") * NUM_CORES + lax.axis_index("c")
    b = wid // W_PER_BATCH
    off = (wid % W_PER_BATCH) * B_PER_W
    half = B_PER_W // 2
    pltpu.sync_copy(idx_hbm.at[b, pl.ds(off, half)], idx_v.at[pl.ds(0, half)])
    rest = pltpu.async_copy(
        idx_hbm.at[b, pl.ds(off + half, half)],
        idx_v.at[pl.ds(half, half)],
        isem,
    )
    wsems = [wsem0, wsem1]
    pend = [None, None]
    for g in range(N_GROUPS):
        s = g % 2
        if g == N_GROUPS // 2:
            rest.wait()
        if pend[s] is not None:
            pend[s].wait()
        gathers = []
        for i in range(GROUP):
            c = g * GROUP + i
            gathers.append(
                pltpu.async_copy(
                    table_hbm.at[idx_v.at[pl.ds(c * CHUNK, CHUNK)]],
                    rows_v.at[s, pl.ds(i * CHUNK, CHUNK)],
                    gsem,
                )
            )
        for ga in gathers:
            ga.wait()
        pend[s] = pltpu.async_copy(
            rows_v.at[s],
            out_hbm.at[b, pl.ds(off + g * G_ROWS, G_ROWS)],
            wsems[s],
        )
    for s in range(2):
        if pend[s] is not None:
            pend[s].wait()


def kernel(input_ids, emb_table):
    return _emb_lookup(input_ids.astype(jnp.int32), emb_table)


# speedup vs baseline: 1.0299x; 1.0299x over previous
import functools

import jax
import jax.numpy as jnp
from jax import lax
from jax.experimental import pallas as pl
from jax.experimental.pallas import tpu as pltpu
from jax.experimental.pallas import tpu_sc as plsc

BATCH = 4
SEQ = 2048
D_MODEL = 768

NUM_CORES = 2
NUM_SUBCORES = 16
NUM_WORKERS = NUM_CORES * NUM_SUBCORES
B_PER_W = BATCH * SEQ // NUM_WORKERS
W_PER_BATCH = SEQ // B_PER_W
CHUNK = 64
N_CHUNKS = B_PER_W // CHUNK

_mesh = plsc.VectorSubcoreMesh(core_axis_name="c", subcore_axis_name="s")


@functools.partial(
    pl.kernel,
    mesh=_mesh,
    out_type=jax.ShapeDtypeStruct((BATCH, SEQ, D_MODEL), jnp.float32),
    scratch_types=[
        pltpu.VMEM((B_PER_W,), jnp.int32),
        pltpu.VMEM((2, CHUNK, D_MODEL), jnp.float32),
        pltpu.SemaphoreType.DMA,
        pltpu.SemaphoreType.DMA,
        pltpu.SemaphoreType.DMA,
        pltpu.SemaphoreType.DMA,
        pltpu.SemaphoreType.DMA,
    ],
)
def _emb_lookup(
    idx_hbm, table_hbm, out_hbm, idx_v, rows_v, gsem0, gsem1, wsem0, wsem1, isem
):
    wid = lax.axis_index("s") * NUM_CORES + lax.axis_index("c")
    b = wid // W_PER_BATCH
    off = (wid % W_PER_BATCH) * B_PER_W
    half = B_PER_W // 2
    pltpu.sync_copy(idx_hbm.at[b, pl.ds(off, half)], idx_v.at[pl.ds(0, half)])
    rest = pltpu.async_copy(
        idx_hbm.at[b, pl.ds(off + half, half)],
        idx_v.at[pl.ds(half, half)],
        isem,
    )
    gsems = [gsem0, gsem1]
    wsems = [wsem0, wsem1]
    gathers = [None] * N_CHUNKS
    writes = [None] * N_CHUNKS
    for c in range(N_CHUNKS):
        if c == N_CHUNKS // 2:
            rest.wait()
        if c >= 2:
            writes[c - 2].wait()
        gathers[c] = pltpu.async_copy(
            table_hbm.at[idx_v.at[pl.ds(c * CHUNK, CHUNK)]],
            rows_v.at[c % 2],
            gsems[c % 2],
        )
        if c >= 1:
            p = c - 1
            gathers[p].wait()
            writes[p] = pltpu.async_copy(
                rows_v.at[p % 2],
                out_hbm.at[b, pl.ds(off + p * CHUNK, CHUNK)],
                wsems[p % 2],
            )
    last = N_CHUNKS - 1
    gathers[last].wait()
    writes[last] = pltpu.async_copy(
        rows_v.at[last % 2],
        out_hbm.at[b, pl.ds(off + last * CHUNK, CHUNK)],
        wsems[last % 2],
    )
    writes[last - 1].wait()
    writes[last].wait()


def kernel(input_ids, emb_table):
    return _emb_lookup(input_ids.astype(jnp.int32), emb_table)
